# baseline (speedup 1.0000x reference)
import functools

import jax
import jax.numpy as jnp
from jax import lax
from jax.experimental import pallas as pl
from jax.experimental.pallas import tpu as pltpu
from jax.experimental.pallas import tpu_sc as plsc

_EMBED = 256
_H = 8
_P = 16
_D = 32
_BS = 16
_Q = 300
_LEN_V = 5440
_SLOTS = 4 * _H * _P
_NC, _NS = 2, 16
_NW = _NC * _NS
_ITEMS = _BS * _Q
_PER_W = _ITEMS // _NW


def _heads_body(q_ref, rp_ref, wox_ref, woy_ref, wa_ref, box_ref, boy_ref,
                ba_ref, idx_ref, wgt_ref):
    b = pl.program_id(0)
    q = q_ref[0]
    offx = jnp.dot(q, wox_ref[...], preferred_element_type=jnp.float32) + box_ref[...]
    offy = jnp.dot(q, woy_ref[...], preferred_element_type=jnp.float32) + boy_ref[...]
    al = jnp.dot(q, wa_ref[...], preferred_element_type=jnp.float32) + ba_ref[...]
    e = jnp.exp(al)
    bi = lax.broadcasted_iota(jnp.int32, (_H * _P, _H * _P), 0) // _P
    bj = lax.broadcasted_iota(jnp.int32, (_H * _P, _H * _P), 1) // _P
    blockdiag = (bi == bj).astype(jnp.float32)
    attnw = e / jnp.dot(e, blockdiag, preferred_element_type=jnp.float32)

    lane = lax.broadcasted_iota(jnp.int32, (1, _H * _P), 1)
    p = lane % _P
    lvl = p // 4
    wl = jnp.int32(64) >> lvl
    vstart = jnp.where(lvl == 0, 0,
                       jnp.where(lvl == 1, 4096,
                                 jnp.where(lvl == 2, 5120, 5376)))
    hlane = lane // _P
    wf = wl.astype(jnp.float32)

    rp = rp_ref[0]
    cx, cy = rp[:, 0:1], rp[:, 1:2]
    rw, rh = rp[:, 2:3], rp[:, 3:4]
    x = (cx + offx * 0.125 * rw) * wf - 0.5
    y = (cy + offy * 0.125 * rh) * wf - 0.5
    x0 = jnp.floor(x)
    y0 = jnp.floor(y)
    wx1 = x - x0
    wx0 = 1.0 - wx1
    wy1 = y - y0
    wy0 = 1.0 - wy1
    rows = []
    wgts = []
    vbase = (b * _LEN_V) * _H
    for dx, dy in ((0, 0), (1, 0), (0, 1), (1, 1)):
        xc = x0 + dx
        yc = y0 + dy
        valid = ((xc >= 0.0) & (xc <= wf - 1.0)
                 & (yc >= 0.0) & (yc <= wf - 1.0))
        xi = jnp.clip(xc, 0.0, wf - 1.0).astype(jnp.int32)
        yi = jnp.clip(yc, 0.0, wf - 1.0).astype(jnp.int32)
        pos = vstart + yi * wl + xi
        rows.append(vbase + pos * _H + hlane)
        wx = wx1 if dx else wx0
        wy = wy1 if dy else wy0
        wgts.append(wx * wy * attnw * valid.astype(jnp.float32))
    idx_ref[0] = jnp.concatenate(rows, axis=-1)
    wgt_ref[0] = jnp.concatenate(wgts, axis=-1)


def _run_heads(query, rp, wox, woy, wa, box, boy, ba):
    return pl.pallas_call(
        _heads_body,
        grid=(_BS,),
        in_specs=[
            pl.BlockSpec((1, _Q, _EMBED), lambda b: (b, 0, 0)),
            pl.BlockSpec((1, _Q, 4), lambda b: (b, 0, 0)),
            pl.BlockSpec((_EMBED, _H * _P), lambda b: (0, 0)),
            pl.BlockSpec((_EMBED, _H * _P), lambda b: (0, 0)),
            pl.BlockSpec((_EMBED, _H * _P), lambda b: (0, 0)),
            pl.BlockSpec((1, _H * _P), lambda b: (0, 0)),
            pl.BlockSpec((1, _H * _P), lambda b: (0, 0)),
            pl.BlockSpec((1, _H * _P), lambda b: (0, 0)),
        ],
        out_specs=[
            pl.BlockSpec((1, _Q, _SLOTS), lambda b: (b, 0, 0)),
            pl.BlockSpec((1, _Q, _SLOTS), lambda b: (b, 0, 0)),
        ],
        out_shape=[
            jax.ShapeDtypeStruct((_BS, _Q, _SLOTS), jnp.int32),
            jax.ShapeDtypeStruct((_BS, _Q, _SLOTS), jnp.float32),
        ],
    )(query, rp, wox, woy, wa, box, boy, ba)


def _sc_gather(idx, wgt, val_flat):
    mesh = plsc.VectorSubcoreMesh(core_axis_name="c", subcore_axis_name="s")

    chunk = 6
    nchunks = _PER_W // chunk

    @functools.partial(
        pl.kernel,
        mesh=mesh,
        compiler_params=pltpu.CompilerParams(use_tc_tiling_on_sc=False),
        out_type=jax.ShapeDtypeStruct((_BS, _Q, _H * _D), jnp.float32),
        scratch_types=[
            pltpu.VMEM((2, chunk, _SLOTS), jnp.int32),
            pltpu.VMEM((2, chunk, _SLOTS), jnp.float32),
            pltpu.VMEM((3, _SLOTS, _D), jnp.float32),
            pltpu.VMEM((_PER_W, _H * _D), jnp.float32),
            pltpu.SemaphoreType.DMA,
            pltpu.SemaphoreType.DMA,
            pltpu.SemaphoreType.DMA,
            pltpu.SemaphoreType.DMA,
        ],
    )
    def k(idx_hbm, wgt_hbm, val_hbm, out_hbm, idx_v, w_v, rows_v, out_v,
          gsem0, gsem1, gsem2, csem):
        gsems = (gsem0, gsem1, gsem2)
        wid = lax.axis_index("s") * _NC + lax.axis_index("c")
        b_w = wid // 2
        q0 = (wid % 2) * _PER_W

        def issue_chunk(kk):
            cb = kk % 2
            pltpu.async_copy(
                idx_hbm.at[b_w, pl.ds(q0 + kk * chunk, chunk)],
                idx_v.at[cb], csem)
            pltpu.async_copy(
                wgt_hbm.at[b_w, pl.ds(q0 + kk * chunk, chunk)],
                w_v.at[cb], csem)

        def wait_chunk():
            pltpu.make_async_copy(
                idx_hbm.at[0, pl.ds(0, chunk)], idx_v.at[0], csem).wait()
            pltpu.make_async_copy(
                wgt_hbm.at[0, pl.ds(0, chunk)], w_v.at[0], csem).wait()

        def load_chunk(kk):
            issue_chunk(kk)
            wait_chunk()

        def fire(i, rb):
            sem = gsems[rb]
            cb = (i // chunk) % 2
            j = i % chunk
            for c in range(4):
                pltpu.async_copy(
                    val_hbm.at[idx_v.at[cb, j, pl.ds(c * 128, 128)]],
                    rows_v.at[rb, pl.ds(c * 128, 128)], sem)

        def drain(rb):
            sem = gsems[rb]
            for c in range(4):
                pltpu.make_async_copy(
                    val_hbm.at[idx_v.at[0, 0, pl.ds(c * 128, 128)]],
                    rows_v.at[rb, pl.ds(c * 128, 128)], sem).wait()

        splats = [jnp.full((16, 1), p, jnp.int32) for p in range(_P)]
        splat_dn = lax.GatherDimensionNumbers(
            offset_dims=(), collapsed_slice_dims=(0,), start_index_map=(0,))

        def compute(i, rb):
            cb = (i // chunk) % 2
            j = i % chunk
            for h in range(_H):
                z = jnp.zeros((16,), jnp.float32)

                @plsc.parallel_loop(0, 4, unroll=4, carry=(z, z))
                def accs(c, accs_c, h=h):
                    a0, a1 = accs_c
                    g = c * 128 + h * _P
                    wv = w_v[cb, j, pl.ds(g, _P)]
                    for p in range(_P):
                        wgt_s = lax.gather(
                            wv, splats[p], splat_dn, slice_sizes=(1,),
                            mode=lax.GatherScatterMode.PROMISE_IN_BOUNDS)
                        a0 = a0 + wgt_s * rows_v[rb, g + p, pl.ds(0, 16)]
                        a1 = a1 + wgt_s * rows_v[rb, g + p, pl.ds(16, 16)]
                    return (a0, a1)

                a0, a1 = accs
                out_v[i, pl.ds(h * 32, 16)] = a0
                out_v[i, pl.ds(h * 32 + 16, 16)] = a1

        load_chunk(jnp.int32(0))
        fire(jnp.int32(0), 0)
        fire(jnp.int32(1), 1)
        load_chunk(jnp.int32(1))

        def triple(tt, carry):
            i0 = 3 * tt
            i1 = i0 + 1
            i2 = i0 + 2
            fire(i2, 2)
            drain(0)
            compute(i0, 0)

            @pl.when(i0 + 3 < _PER_W)
            def _():
                @pl.when(jnp.logical_and((i0 + 3) % chunk == 0,
                                         (i0 + 3) // chunk >= 2))
                def _():
                    wait_chunk()

                fire(i0 + 3, 0)

            drain(1)
            compute(i1, 1)

            @pl.when(i1 + 3 < _PER_W)
            def _():
                fire(i1 + 3, 1)

            drain(2)
            compute(i2, 2)

            @pl.when(jnp.logical_and((i2 + 1) % chunk == 0,
                                     (i2 + 1) // chunk + 1 < nchunks))
            def _():
                issue_chunk((i2 + 1) // chunk + 1)

            return carry

        lax.fori_loop(0, _PER_W // 3, triple, 0)
        pltpu.sync_copy(out_v, out_hbm.at[b_w, pl.ds(q0, _PER_W)])

    return k(idx, wgt, val_flat)


def kernel(query, reference_points, value, value_spatial_shapes, W_off, b_off,
           W_attn, b_attn):
    del value_spatial_shapes
    rp = reference_points.reshape(_BS, _Q, 4)
    wox = W_off[0::2, :].T
    woy = W_off[1::2, :].T
    box = b_off[0::2].reshape(1, _H * _P)
    boy = b_off[1::2].reshape(1, _H * _P)
    wa = W_attn.T
    ba = b_attn.reshape(1, _H * _P)
    idx, wgt = _run_heads(query, rp, wox, woy, wa, box, boy, ba)
    val_flat = value.reshape(_BS * _LEN_V * _H, _D)
    return _sc_gather(idx, wgt, val_flat)


# speedup vs baseline: 1.0117x; 1.0117x over previous
import functools

import jax
import jax.numpy as jnp
from jax import lax
from jax.experimental import pallas as pl
from jax.experimental.pallas import tpu as pltpu
from jax.experimental.pallas import tpu_sc as plsc

_EMBED = 256
_H = 8
_P = 16
_D = 32
_BS = 16
_Q = 300
_LEN_V = 5440
_SLOTS = 4 * _H * _P
_NC, _NS = 2, 16
_NW = _NC * _NS
_ITEMS = _BS * _Q
_PER_W = _ITEMS // _NW


def _heads_body(q_ref, rp_ref, wox_ref, woy_ref, wa_ref, box_ref, boy_ref,
                ba_ref, idx_ref, wgt_ref):
    b = pl.program_id(0)
    q = q_ref[0]
    offx = jnp.dot(q, wox_ref[...], preferred_element_type=jnp.float32) + box_ref[...]
    offy = jnp.dot(q, woy_ref[...], preferred_element_type=jnp.float32) + boy_ref[...]
    al = jnp.dot(q, wa_ref[...], preferred_element_type=jnp.float32) + ba_ref[...]
    e = jnp.exp(al)
    bi = lax.broadcasted_iota(jnp.int32, (_H * _P, _H * _P), 0) // _P
    bj = lax.broadcasted_iota(jnp.int32, (_H * _P, _H * _P), 1) // _P
    blockdiag = (bi == bj).astype(jnp.float32)
    attnw = e / jnp.dot(e, blockdiag, preferred_element_type=jnp.float32)

    lane = lax.broadcasted_iota(jnp.int32, (1, _H * _P), 1)
    p = lane % _P
    lvl = p // 4
    wl = jnp.int32(64) >> lvl
    vstart = jnp.where(lvl == 0, 0,
                       jnp.where(lvl == 1, 4096,
                                 jnp.where(lvl == 2, 5120, 5376)))
    hlane = lane // _P
    wf = wl.astype(jnp.float32)

    rp = rp_ref[0]
    cx, cy = rp[:, 0:1], rp[:, 1:2]
    rw, rh = rp[:, 2:3], rp[:, 3:4]
    x = (cx + offx * 0.125 * rw) * wf - 0.5
    y = (cy + offy * 0.125 * rh) * wf - 0.5
    x0 = jnp.floor(x)
    y0 = jnp.floor(y)
    wx1 = x - x0
    wx0 = 1.0 - wx1
    wy1 = y - y0
    wy0 = 1.0 - wy1
    rows = []
    wgts = []
    vbase = (b * _LEN_V) * _H
    for dx, dy in ((0, 0), (1, 0), (0, 1), (1, 1)):
        xc = x0 + dx
        yc = y0 + dy
        valid = ((xc >= 0.0) & (xc <= wf - 1.0)
                 & (yc >= 0.0) & (yc <= wf - 1.0))
        xi = jnp.clip(xc, 0.0, wf - 1.0).astype(jnp.int32)
        yi = jnp.clip(yc, 0.0, wf - 1.0).astype(jnp.int32)
        pos = vstart + yi * wl + xi
        rows.append(vbase + pos * _H + hlane)
        wx = wx1 if dx else wx0
        wy = wy1 if dy else wy0
        wgts.append(wx * wy * attnw * valid.astype(jnp.float32))
    idx_ref[0] = jnp.concatenate(rows, axis=-1)
    wgt_ref[0] = jnp.concatenate(wgts, axis=-1)


def _run_heads(query, rp, wox, woy, wa, box, boy, ba):
    return pl.pallas_call(
        _heads_body,
        grid=(_BS,),
        in_specs=[
            pl.BlockSpec((1, _Q, _EMBED), lambda b: (b, 0, 0)),
            pl.BlockSpec((1, _Q, 4), lambda b: (b, 0, 0)),
            pl.BlockSpec((_EMBED, _H * _P), lambda b: (0, 0)),
            pl.BlockSpec((_EMBED, _H * _P), lambda b: (0, 0)),
            pl.BlockSpec((_EMBED, _H * _P), lambda b: (0, 0)),
            pl.BlockSpec((1, _H * _P), lambda b: (0, 0)),
            pl.BlockSpec((1, _H * _P), lambda b: (0, 0)),
            pl.BlockSpec((1, _H * _P), lambda b: (0, 0)),
        ],
        out_specs=[
            pl.BlockSpec((1, _Q, _SLOTS), lambda b: (b, 0, 0)),
            pl.BlockSpec((1, _Q, _SLOTS), lambda b: (b, 0, 0)),
        ],
        out_shape=[
            jax.ShapeDtypeStruct((_BS, _Q, _SLOTS), jnp.int32),
            jax.ShapeDtypeStruct((_BS, _Q, _SLOTS), jnp.float32),
        ],
    )(query, rp, wox, woy, wa, box, boy, ba)


def _sc_gather(idx, wgt, val_flat):
    mesh = plsc.VectorSubcoreMesh(core_axis_name="c", subcore_axis_name="s")

    chunk = 6
    nchunks = _PER_W // chunk

    @functools.partial(
        pl.kernel,
        mesh=mesh,
        compiler_params=pltpu.CompilerParams(use_tc_tiling_on_sc=False),
        out_type=jax.ShapeDtypeStruct((_BS, _Q, _H * _D), jnp.float32),
        scratch_types=[
            pltpu.VMEM((2, chunk, _SLOTS), jnp.int32),
            pltpu.VMEM((2, chunk, _SLOTS), jnp.float32),
            pltpu.VMEM((3, _SLOTS, _D), jnp.float32),
            pltpu.VMEM((_PER_W, _H * _D), jnp.float32),
            pltpu.SemaphoreType.DMA,
            pltpu.SemaphoreType.DMA,
            pltpu.SemaphoreType.DMA,
            pltpu.SemaphoreType.DMA,
        ],
    )
    def k(idx_hbm, wgt_hbm, val_hbm, out_hbm, idx_v, w_v, rows_v, out_v,
          gsem0, gsem1, gsem2, csem):
        gsems = (gsem0, gsem1, gsem2)
        wid = lax.axis_index("s") * _NC + lax.axis_index("c")
        b_w = wid // 2
        q0 = (wid % 2) * _PER_W

        def issue_chunk(kk):
            cb = kk % 2
            pltpu.async_copy(
                idx_hbm.at[b_w, pl.ds(q0 + kk * chunk, chunk)],
                idx_v.at[cb], csem)
            pltpu.async_copy(
                wgt_hbm.at[b_w, pl.ds(q0 + kk * chunk, chunk)],
                w_v.at[cb], csem)

        def wait_chunk():
            pltpu.make_async_copy(
                idx_hbm.at[0, pl.ds(0, chunk)], idx_v.at[0], csem).wait()
            pltpu.make_async_copy(
                wgt_hbm.at[0, pl.ds(0, chunk)], w_v.at[0], csem).wait()

        def load_chunk(kk):
            issue_chunk(kk)
            wait_chunk()

        def fire(i, rb):
            sem = gsems[rb]
            cb = (i // chunk) % 2
            j = i % chunk
            for c in range(4):
                pltpu.async_copy(
                    val_hbm.at[idx_v.at[cb, j, pl.ds(c * 128, 128)]],
                    rows_v.at[rb, pl.ds(c * 128, 128)], sem)

        def drain(rb):
            sem = gsems[rb]
            for c in range(4):
                pltpu.make_async_copy(
                    val_hbm.at[idx_v.at[0, 0, pl.ds(c * 128, 128)]],
                    rows_v.at[rb, pl.ds(c * 128, 128)], sem).wait()

        splats = [jnp.full((16, 1), p, jnp.int32) for p in range(_P)]
        splat_dn = lax.GatherDimensionNumbers(
            offset_dims=(), collapsed_slice_dims=(0,), start_index_map=(0,))

        def compute(i, rb):
            cb = (i // chunk) % 2
            j = i % chunk
            for h in range(_H):
                z = jnp.zeros((16,), jnp.float32)

                @plsc.parallel_loop(0, 4, unroll=2, carry=(z, z))
                def accs(c, accs_c, h=h):
                    a0, a1 = accs_c
                    g = c * 128 + h * _P
                    wv = w_v[cb, j, pl.ds(g, _P)]
                    for p in range(_P):
                        wgt_s = lax.gather(
                            wv, splats[p], splat_dn, slice_sizes=(1,),
                            mode=lax.GatherScatterMode.PROMISE_IN_BOUNDS)
                        a0 = a0 + wgt_s * rows_v[rb, g + p, pl.ds(0, 16)]
                        a1 = a1 + wgt_s * rows_v[rb, g + p, pl.ds(16, 16)]
                    return (a0, a1)

                a0, a1 = accs
                out_v[i, pl.ds(h * 32, 16)] = a0
                out_v[i, pl.ds(h * 32 + 16, 16)] = a1

        load_chunk(jnp.int32(0))
        fire(jnp.int32(0), 0)
        fire(jnp.int32(1), 1)
        load_chunk(jnp.int32(1))

        def triple(tt, carry):
            i0 = 3 * tt
            i1 = i0 + 1
            i2 = i0 + 2
            fire(i2, 2)
            drain(0)
            compute(i0, 0)

            @pl.when(i0 + 3 < _PER_W)
            def _():
                @pl.when(jnp.logical_and((i0 + 3) % chunk == 0,
                                         (i0 + 3) // chunk >= 2))
                def _():
                    wait_chunk()

                fire(i0 + 3, 0)

            drain(1)
            compute(i1, 1)

            @pl.when(i1 + 3 < _PER_W)
            def _():
                fire(i1 + 3, 1)

            drain(2)
            compute(i2, 2)

            @pl.when(jnp.logical_and((i2 + 1) % chunk == 0,
                                     (i2 + 1) // chunk + 1 < nchunks))
            def _():
                issue_chunk((i2 + 1) // chunk + 1)

            return carry

        lax.fori_loop(0, _PER_W // 3, triple, 0)
        pltpu.sync_copy(out_v, out_hbm.at[b_w, pl.ds(q0, _PER_W)])

    return k(idx, wgt, val_flat)


def kernel(query, reference_points, value, value_spatial_shapes, W_off, b_off,
           W_attn, b_attn):
    del value_spatial_shapes
    rp = reference_points.reshape(_BS, _Q, 4)
    wox = W_off[0::2, :].T
    woy = W_off[1::2, :].T
    box = b_off[0::2].reshape(1, _H * _P)
    boy = b_off[1::2].reshape(1, _H * _P)
    wa = W_attn.T
    ba = b_attn.reshape(1, _H * _P)
    idx, wgt = _run_heads(query, rp, wox, woy, wa, box, boy, ba)
    val_flat = value.reshape(_BS * _LEN_V * _H, _D)
    return _sc_gather(idx, wgt, val_flat)
